# baseline (speedup 1.0000x reference)
import functools

import jax
import jax.numpy as jnp
import numpy as np
from jax.experimental import pallas as pl
from jax.experimental.pallas import tpu as pltpu
from jax.scipy.stats import norm

B, T, H, Wd = 128, 16, 32, 32
D = H * Wd
HID = 2 * D
E = 16
NOISE_EPS = 1e-2
LOSS_COEF = 1e-2
P = 256
TILE = 8


def _ffn_body(po_ref, nt_ref, xs_ref, w1_ref, b1_ref, w2_ref, b2_ref, out_ref):
    m = pl.program_id(0)
    e = pl.program_id(1)
    po = po_ref[m, e]
    nt = nt_ref[m, e]
    w1 = w1_ref[0, 0]
    w2 = w2_ref[0, 0]
    b1 = b1_ref[0, 0, 0]
    b2 = b2_ref[0, 0, 0]

    def tile_step(t, carry):
        base = pl.multiple_of(po + t * TILE, TILE)
        xt = xs_ref[0, pl.ds(base, TILE), :]
        h = jnp.dot(xt, w1, preferred_element_type=jnp.float32) + b1[None, :]
        h = jnp.maximum(h, 0.0)
        yt = jnp.dot(h, w2, preferred_element_type=jnp.float32) + b2[None, :]
        out_ref[0, pl.ds(base, TILE), :] = yt
        return carry

    jax.lax.fori_loop(0, nt, tile_step, 0)


def _grouped_ffn(po, nt, xs, W1, b1, W2, b2):
    grid_spec = pltpu.PrefetchScalarGridSpec(
        num_scalar_prefetch=2,
        grid=(3, E),
        in_specs=[
            pl.BlockSpec((1, P, D), lambda m, e, po, nt: (m, 0, 0)),
            pl.BlockSpec((1, 1, D, HID), lambda m, e, po, nt: (m, e, 0, 0)),
            pl.BlockSpec((1, 1, 1, HID), lambda m, e, po, nt: (m, e, 0, 0)),
            pl.BlockSpec((1, 1, HID, D), lambda m, e, po, nt: (m, e, 0, 0)),
            pl.BlockSpec((1, 1, 1, D), lambda m, e, po, nt: (m, e, 0, 0)),
        ],
        out_specs=pl.BlockSpec((1, P, D), lambda m, e, po, nt: (m, 0, 0)),
    )
    return pl.pallas_call(
        _ffn_body,
        grid_spec=grid_spec,
        out_shape=jax.ShapeDtypeStruct((3, P, D), jnp.float32),
        compiler_params=pltpu.CompilerParams(
            dimension_semantics=("arbitrary", "arbitrary"),
        ),
    )(po, nt, xs, W1, b1, W2, b2)


def _conv3d(x, w, b, padding):
    out = jax.lax.conv_general_dilated(
        x, w, window_strides=(1, 1, 1), padding=padding,
        dimension_numbers=('NCDHW', 'OIDHW', 'NCDHW'))
    return out + b[None, :, None, None, None]


def _cv_squared(x):
    eps = 1e-10
    return jnp.var(x, ddof=1) / (jnp.mean(x) ** 2 + eps)


def kernel(input, conv1_w, conv1_b, conv2_w, conv2_b, w_gate, w_noise, W1, b1, W2, b2):
    x = jnp.transpose(input, (0, 2, 1, 3, 4))
    x = jax.nn.relu(_conv3d(x, conv1_w, conv1_b, [(1, 1), (1, 1), (1, 1)]))
    x = jax.nn.relu(_conv3d(x, conv2_w, conv2_b, [(0, 0), (1, 1), (1, 1)]))
    xf = x.reshape(B, -1)

    keys = jax.random.split(jax.random.key(42), 3)
    noise = jnp.stack([jax.random.normal(k, (B, E), dtype=jnp.float32) for k in keys])

    clean = jnp.einsum('bd,mde->mbe', xf, w_gate)
    raw = jnp.einsum('bd,mde->mbe', xf, w_noise)
    stddev = jax.nn.softplus(raw) + NOISE_EPS
    noisy = clean + noise * stddev

    v0 = jnp.max(noisy, axis=-1)
    e0 = jnp.argmax(noisy, axis=-1).astype(jnp.int32)
    masked = jnp.where(jax.nn.one_hot(e0, E, dtype=bool), -jnp.inf, noisy)
    v1 = jnp.max(masked, axis=-1)

    is_in = noisy > v1[..., None]
    p_in = norm.cdf((clean - v1[..., None]) / stddev)
    p_out = norm.cdf((clean - v0[..., None]) / stddev)
    load = jnp.where(is_in, p_in, p_out).sum(axis=1)
    importance = jax.nn.one_hot(e0, E, dtype=jnp.float32).sum(axis=1)
    aux = jnp.float32(0.0)
    for m in range(3):
        aux = aux + (_cv_squared(importance[m]) + _cv_squared(load[m])) * LOSS_COEF

    counts = importance.astype(jnp.int32)
    ntiles = (counts + TILE - 1) // TILE
    po = (jnp.cumsum(ntiles, axis=-1) - ntiles) * TILE
    so = jnp.cumsum(counts, axis=-1) - counts
    sort_idx = jnp.argsort(e0, axis=-1).astype(jnp.int32)
    es = jnp.take_along_axis(e0, sort_idx, axis=-1)
    rank = jnp.arange(B)[None, :] - jnp.take_along_axis(so, es, axis=-1)
    dst = jnp.take_along_axis(po, es, axis=-1) + rank
    ids_padded = jnp.zeros((3, P), jnp.int32)
    ids_padded = jax.vmap(lambda i, s, d: i.at[d].set(s))(ids_padded, sort_idx, dst)
    pos = jax.vmap(lambda p, s, d: p.at[s].set(d))(
        jnp.zeros((3, B), jnp.int32), sort_idx, dst)

    xs = jnp.take(xf, ids_padded.reshape(-1), axis=0).reshape(3, P, D)
    ys = _grouped_ffn(po.astype(jnp.int32), ntiles.astype(jnp.int32),
                      xs, W1, b1.reshape(3, E, 1, HID), W2, b2.reshape(3, E, 1, D))
    y = jnp.take_along_axis(ys, pos[..., None], axis=1)

    funcs = y.reshape(3, B, 1, 1, H, Wd)
    transform_f, add_f, quad_f = funcs[0], funcs[1], funcs[2]
    output = input * input * quad_f + input * transform_f + add_f
    output = jax.nn.sigmoid(output)
    return output, aux


# speedup vs baseline: 1.1364x; 1.1364x over previous
import functools

import jax
import jax.numpy as jnp
import numpy as np
from jax.experimental import pallas as pl
from jax.experimental.pallas import tpu as pltpu
from jax.scipy.stats import norm

B, T, H, Wd = 128, 16, 32, 32
WP = Wd + 2
HWP = H * WP
LPAD = 40
LT = HWP + 2 * LPAD
_DELTAS = [WP * (dh - 1) + (dw - 1) for dh in range(3) for dw in range(3)]
_CB = 8
D = H * Wd
HID = 2 * D
E = 16
NOISE_EPS = 1e-2
LOSS_COEF = 1e-2
P = 256
TILE = 8


_idx = [(co * 16 + tt, (3 * dh + dw) * 18 + tt + dt, co, dt, dh, dw)
        for co in range(10) for tt in range(16)
        for dh in range(3) for dw in range(3) for dt in range(3)]
_C1_ROW = np.array([i[0] for i in _idx], np.int32)
_C1_COL = np.array([i[1] for i in _idx], np.int32)
_C1_CO = np.array([i[2] for i in _idx], np.int32)
_C1_DT = np.array([i[3] for i in _idx], np.int32)
_C1_DH = np.array([i[4] for i in _idx], np.int32)
_C1_DW = np.array([i[5] for i in _idx], np.int32)
_WMASK = np.zeros((1, HWP), np.float32)
for _r in range(HWP):
    if 1 <= _r % WP <= Wd:
        _WMASK[0, _r] = 1.0
del _idx, _r


def _conv_body(a_ref, w1_ref, b1_ref, w2_ref, wm_ref, b2_ref, out_ref):
    w1mt = w1_ref[:]
    b1col = b1_ref[:]
    w2gt = w2_ref[:]
    wmask = wm_ref[:]
    b2 = b2_ref[0, 0]
    for i in range(_CB):
        a = a_ref[i]
        pt = jnp.concatenate(
            [a[:, LPAD + d:LPAD + d + HWP] for d in _DELTAS], axis=0)
        r = jnp.dot(w1mt, pt, preferred_element_type=jnp.float32) + b1col
        r = jnp.maximum(r, 0.0)
        g = jnp.dot(w2gt, r, preferred_element_type=jnp.float32) * wmask
        gp = jnp.pad(g, ((0, 0), (LPAD, LPAD)))
        acc = gp[0:1, LPAD + _DELTAS[0]:LPAD + _DELTAS[0] + HWP]
        for s in range(1, 9):
            acc = acc + gp[s:s + 1, LPAD + _DELTAS[s]:LPAD + _DELTAS[s] + HWP]
        out_ref[i, :] = jnp.maximum(acc + b2, 0.0)[0]


def _conv_front(a_t, w1mt, b1col, w2gt, wmask, b2s):
    return pl.pallas_call(
        _conv_body,
        grid=(B // _CB,),
        in_specs=[
            pl.BlockSpec((_CB, 18, LT), lambda i: (i, 0, 0)),
            pl.BlockSpec((160, 162), lambda i: (0, 0)),
            pl.BlockSpec((160, 1), lambda i: (0, 0)),
            pl.BlockSpec((9, 160), lambda i: (0, 0)),
            pl.BlockSpec((1, HWP), lambda i: (0, 0)),
            pl.BlockSpec((1, 1), lambda i: (0, 0), memory_space=pltpu.SMEM),
        ],
        out_specs=pl.BlockSpec((_CB, HWP), lambda i: (i, 0)),
        out_shape=jax.ShapeDtypeStruct((B, HWP), jnp.float32),
    )(a_t, w1mt, b1col, w2gt, wmask, b2s)


def _ffn_body(po_ref, nt_ref, xs_ref, w1_ref, b1_ref, w2_ref, b2_ref, out_ref):
    m = pl.program_id(0)
    e = pl.program_id(1)
    po = po_ref[m, e]
    nt = nt_ref[m, e]
    w1 = w1_ref[0, 0]
    w2 = w2_ref[0, 0]
    b1 = b1_ref[0, 0, 0]
    b2 = b2_ref[0, 0, 0]

    def tile_step(t, carry):
        base = pl.multiple_of(po + t * TILE, TILE)
        xt = xs_ref[0, pl.ds(base, TILE), :]
        h = jnp.dot(xt, w1, preferred_element_type=jnp.float32) + b1[None, :]
        h = jnp.maximum(h, 0.0)
        yt = jnp.dot(h, w2, preferred_element_type=jnp.float32) + b2[None, :]
        out_ref[0, pl.ds(base, TILE), :] = yt
        return carry

    jax.lax.fori_loop(0, nt, tile_step, 0)


def _grouped_ffn(po, nt, xs, W1, b1, W2, b2):
    grid_spec = pltpu.PrefetchScalarGridSpec(
        num_scalar_prefetch=2,
        grid=(3, E),
        in_specs=[
            pl.BlockSpec((1, P, D), lambda m, e, po, nt: (m, 0, 0)),
            pl.BlockSpec((1, 1, D, HID), lambda m, e, po, nt: (m, e, 0, 0)),
            pl.BlockSpec((1, 1, 1, HID), lambda m, e, po, nt: (m, e, 0, 0)),
            pl.BlockSpec((1, 1, HID, D), lambda m, e, po, nt: (m, e, 0, 0)),
            pl.BlockSpec((1, 1, 1, D), lambda m, e, po, nt: (m, e, 0, 0)),
        ],
        out_specs=pl.BlockSpec((1, P, D), lambda m, e, po, nt: (m, 0, 0)),
    )
    return pl.pallas_call(
        _ffn_body,
        grid_spec=grid_spec,
        out_shape=jax.ShapeDtypeStruct((3, P, D), jnp.float32),
        compiler_params=pltpu.CompilerParams(
            dimension_semantics=("arbitrary", "arbitrary"),
        ),
    )(po, nt, xs, W1, b1, W2, b2)


def _conv3d(x, w, b, padding):
    out = jax.lax.conv_general_dilated(
        x, w, window_strides=(1, 1, 1), padding=padding,
        dimension_numbers=('NCDHW', 'OIDHW', 'NCDHW'))
    return out + b[None, :, None, None, None]


def _cv_squared(x):
    eps = 1e-10
    return jnp.var(x, ddof=1) / (jnp.mean(x) ** 2 + eps)


def kernel(input, conv1_w, conv1_b, conv2_w, conv2_b, w_gate, w_noise, W1, b1, W2, b2):
    x_raw = input.reshape(B, T, H, Wd)
    a_t = jnp.pad(x_raw, ((0, 0), (1, 1), (0, 0), (1, 1)))
    a_t = jnp.pad(a_t.reshape(B, 18, HWP), ((0, 0), (0, 0), (LPAD, LPAD)))
    w1mt = jnp.zeros((160, 162), jnp.float32).at[_C1_ROW, _C1_COL].set(
        conv1_w[_C1_CO, 0, _C1_DT, _C1_DH, _C1_DW])
    b1col = jnp.repeat(conv1_b, 16).reshape(160, 1)
    w2gt = jnp.transpose(conv2_w[0], (2, 3, 0, 1)).reshape(9, 160)
    out_p = _conv_front(a_t, w1mt, b1col, w2gt, jnp.asarray(_WMASK),
                        conv2_b.reshape(1, 1))
    xf = out_p.reshape(B, H, WP)[:, :, 1:1 + Wd].reshape(B, D)

    keys = jax.random.split(jax.random.key(42), 3)
    noise = jnp.stack([jax.random.normal(k, (B, E), dtype=jnp.float32) for k in keys])

    clean = jnp.einsum('bd,mde->mbe', xf, w_gate)
    raw = jnp.einsum('bd,mde->mbe', xf, w_noise)
    stddev = jax.nn.softplus(raw) + NOISE_EPS
    noisy = clean + noise * stddev

    v0 = jnp.max(noisy, axis=-1)
    e0 = jnp.argmax(noisy, axis=-1).astype(jnp.int32)
    masked = jnp.where(jax.nn.one_hot(e0, E, dtype=bool), -jnp.inf, noisy)
    v1 = jnp.max(masked, axis=-1)

    is_in = noisy > v1[..., None]
    p_in = norm.cdf((clean - v1[..., None]) / stddev)
    p_out = norm.cdf((clean - v0[..., None]) / stddev)
    load = jnp.where(is_in, p_in, p_out).sum(axis=1)
    importance = jax.nn.one_hot(e0, E, dtype=jnp.float32).sum(axis=1)
    aux = jnp.float32(0.0)
    for m in range(3):
        aux = aux + (_cv_squared(importance[m]) + _cv_squared(load[m])) * LOSS_COEF

    counts = importance.astype(jnp.int32)
    ntiles = (counts + TILE - 1) // TILE
    po = (jnp.cumsum(ntiles, axis=-1) - ntiles) * TILE
    so = jnp.cumsum(counts, axis=-1) - counts
    sort_idx = jnp.argsort(e0, axis=-1).astype(jnp.int32)
    es = jnp.take_along_axis(e0, sort_idx, axis=-1)
    rank = jnp.arange(B)[None, :] - jnp.take_along_axis(so, es, axis=-1)
    dst = jnp.take_along_axis(po, es, axis=-1) + rank
    ids_padded = jnp.zeros((3, P), jnp.int32)
    ids_padded = jax.vmap(lambda i, s, d: i.at[d].set(s))(ids_padded, sort_idx, dst)
    pos = jax.vmap(lambda p, s, d: p.at[s].set(d))(
        jnp.zeros((3, B), jnp.int32), sort_idx, dst)

    xs = jnp.take(xf, ids_padded.reshape(-1), axis=0).reshape(3, P, D)
    ys = _grouped_ffn(po.astype(jnp.int32), ntiles.astype(jnp.int32),
                      xs, W1, b1.reshape(3, E, 1, HID), W2, b2.reshape(3, E, 1, D))
    y = jnp.take_along_axis(ys, pos[..., None], axis=1)

    funcs = y.reshape(3, B, 1, 1, H, Wd)
    transform_f, add_f, quad_f = funcs[0], funcs[1], funcs[2]
    output = input * input * quad_f + input * transform_f + add_f
    output = jax.nn.sigmoid(output)
    return output, aux


# speedup vs baseline: 1.1964x; 1.0528x over previous
import functools

import jax
import jax.numpy as jnp
import numpy as np
from jax.experimental import pallas as pl
from jax.experimental.pallas import tpu as pltpu
from jax.scipy.stats import norm

B, T, H, Wd = 128, 16, 32, 32
LPAD = 128
LT = H * Wd + 2 * LPAD
_DELTAS = [Wd * (dh - 1) + (dw - 1) for dh in range(3) for dw in range(3)]
_DWS = [dw for dh in range(3) for dw in range(3)]
_CB = 8
D = H * Wd
HID = 2 * D
E = 16
NOISE_EPS = 1e-2
LOSS_COEF = 1e-2
P = 256
TILE = 8


_idx = [(co * 16 + tt, (3 * dh + dw) * 18 + tt + dt, co, dt, dh, dw)
        for co in range(10) for tt in range(16)
        for dh in range(3) for dw in range(3) for dt in range(3)]
_C1_ROW = np.array([i[0] for i in _idx], np.int32)
_C1_COL = np.array([i[1] for i in _idx], np.int32)
_C1_CO = np.array([i[2] for i in _idx], np.int32)
_C1_DT = np.array([i[3] for i in _idx], np.int32)
_C1_DH = np.array([i[4] for i in _idx], np.int32)
_C1_DW = np.array([i[5] for i in _idx], np.int32)
_WMASK = np.zeros((3, H * Wd), np.float32)
for _dw in range(3):
    for _r in range(H * Wd):
        if 0 <= (_r % Wd) + _dw - 1 < Wd:
            _WMASK[_dw, _r] = 1.0
del _idx, _dw, _r


def _conv_body(a_ref, w1_ref, b1_ref, w2_ref, wm_ref, b2_ref, out_ref):
    w1mt = w1_ref[:]
    b1col = b1_ref[:]
    w2gt = w2_ref[:]
    b2 = b2_ref[0, 0]
    masks = [wm_ref[dw:dw + 1, :] for dw in range(3)]
    for i in range(_CB):
        a = a_ref[i]
        parts = []
        for s in range(9):
            sl = a[:, LPAD + _DELTAS[s]:LPAD + _DELTAS[s] + D]
            if _DWS[s] != 1:
                sl = sl * masks[_DWS[s]]
            parts.append(sl)
        pt = jnp.concatenate(parts, axis=0)
        r = jnp.dot(w1mt, pt, preferred_element_type=jnp.float32) + b1col
        r = jnp.maximum(r, 0.0)
        g = jnp.dot(w2gt, r, preferred_element_type=jnp.float32)
        gp = jnp.pad(g, ((0, 0), (LPAD, LPAD)))
        acc = None
        for s in range(9):
            term = gp[s:s + 1, LPAD + _DELTAS[s]:LPAD + _DELTAS[s] + D]
            if _DWS[s] != 1:
                term = term * masks[_DWS[s]]
            acc = term if acc is None else acc + term
        out_ref[i, :] = jnp.maximum(acc + b2, 0.0)[0]


def _conv_front(a_t, w1mt, b1col, w2gt, wmask, b2s):
    return pl.pallas_call(
        _conv_body,
        grid=(B // _CB,),
        in_specs=[
            pl.BlockSpec((_CB, 18, LT), lambda i: (i, 0, 0)),
            pl.BlockSpec((160, 162), lambda i: (0, 0)),
            pl.BlockSpec((160, 1), lambda i: (0, 0)),
            pl.BlockSpec((9, 160), lambda i: (0, 0)),
            pl.BlockSpec((3, D), lambda i: (0, 0)),
            pl.BlockSpec((1, 1), lambda i: (0, 0), memory_space=pltpu.SMEM),
        ],
        out_specs=pl.BlockSpec((_CB, D), lambda i: (i, 0)),
        out_shape=jax.ShapeDtypeStruct((B, D), jnp.float32),
    )(a_t, w1mt, b1col, w2gt, wmask, b2s)


def _ffn_body(po_ref, nt_ref, xs_ref, w1_ref, b1_ref, w2_ref, b2_ref, out_ref):
    m = pl.program_id(0)
    e = pl.program_id(1)
    po = po_ref[m, e]
    nt = nt_ref[m, e]
    w1 = w1_ref[0, 0]
    w2 = w2_ref[0, 0]
    b1 = b1_ref[0, 0, 0]
    b2 = b2_ref[0, 0, 0]

    def tile_step(t, carry):
        base = pl.multiple_of(po + t * TILE, TILE)
        xt = xs_ref[0, pl.ds(base, TILE), :]
        h = jnp.dot(xt, w1, preferred_element_type=jnp.float32) + b1[None, :]
        h = jnp.maximum(h, 0.0)
        yt = jnp.dot(h, w2, preferred_element_type=jnp.float32) + b2[None, :]
        out_ref[0, pl.ds(base, TILE), :] = yt
        return carry

    jax.lax.fori_loop(0, nt, tile_step, 0)


def _grouped_ffn(po, nt, xs, W1, b1, W2, b2):
    grid_spec = pltpu.PrefetchScalarGridSpec(
        num_scalar_prefetch=2,
        grid=(3, E),
        in_specs=[
            pl.BlockSpec((1, P, D), lambda m, e, po, nt: (m, 0, 0)),
            pl.BlockSpec((1, 1, D, HID), lambda m, e, po, nt: (m, e, 0, 0)),
            pl.BlockSpec((1, 1, 1, HID), lambda m, e, po, nt: (m, e, 0, 0)),
            pl.BlockSpec((1, 1, HID, D), lambda m, e, po, nt: (m, e, 0, 0)),
            pl.BlockSpec((1, 1, 1, D), lambda m, e, po, nt: (m, e, 0, 0)),
        ],
        out_specs=pl.BlockSpec((1, P, D), lambda m, e, po, nt: (m, 0, 0)),
    )
    return pl.pallas_call(
        _ffn_body,
        grid_spec=grid_spec,
        out_shape=jax.ShapeDtypeStruct((3, P, D), jnp.float32),
        compiler_params=pltpu.CompilerParams(
            dimension_semantics=("arbitrary", "arbitrary"),
        ),
    )(po, nt, xs, W1, b1, W2, b2)


def _conv3d(x, w, b, padding):
    out = jax.lax.conv_general_dilated(
        x, w, window_strides=(1, 1, 1), padding=padding,
        dimension_numbers=('NCDHW', 'OIDHW', 'NCDHW'))
    return out + b[None, :, None, None, None]


def _cv_squared(x):
    eps = 1e-10
    return jnp.var(x, ddof=1) / (jnp.mean(x) ** 2 + eps)


def kernel(input, conv1_w, conv1_b, conv2_w, conv2_b, w_gate, w_noise, W1, b1, W2, b2):
    a_t = jnp.pad(input.reshape(B, T, D), ((0, 0), (1, 1), (LPAD, LPAD)))
    w1mt = jnp.zeros((160, 162), jnp.float32).at[_C1_ROW, _C1_COL].set(
        conv1_w[_C1_CO, 0, _C1_DT, _C1_DH, _C1_DW])
    b1col = jnp.repeat(conv1_b, 16).reshape(160, 1)
    w2gt = jnp.transpose(conv2_w[0], (2, 3, 0, 1)).reshape(9, 160)
    xf = _conv_front(a_t, w1mt, b1col, w2gt, jnp.asarray(_WMASK),
                     conv2_b.reshape(1, 1))

    keys = jax.random.split(jax.random.key(42), 3)
    noise = jnp.stack([jax.random.normal(k, (B, E), dtype=jnp.float32) for k in keys])

    clean = jnp.einsum('bd,mde->mbe', xf, w_gate)
    raw = jnp.einsum('bd,mde->mbe', xf, w_noise)
    stddev = jax.nn.softplus(raw) + NOISE_EPS
    noisy = clean + noise * stddev

    v0 = jnp.max(noisy, axis=-1)
    e0 = jnp.argmax(noisy, axis=-1).astype(jnp.int32)
    masked = jnp.where(jax.nn.one_hot(e0, E, dtype=bool), -jnp.inf, noisy)
    v1 = jnp.max(masked, axis=-1)

    is_in = noisy > v1[..., None]
    p_in = norm.cdf((clean - v1[..., None]) / stddev)
    p_out = norm.cdf((clean - v0[..., None]) / stddev)
    load = jnp.where(is_in, p_in, p_out).sum(axis=1)
    importance = jax.nn.one_hot(e0, E, dtype=jnp.float32).sum(axis=1)
    aux = jnp.float32(0.0)
    for m in range(3):
        aux = aux + (_cv_squared(importance[m]) + _cv_squared(load[m])) * LOSS_COEF

    counts = importance.astype(jnp.int32)
    ntiles = (counts + TILE - 1) // TILE
    po = (jnp.cumsum(ntiles, axis=-1) - ntiles) * TILE
    so = jnp.cumsum(counts, axis=-1) - counts
    sort_idx = jnp.argsort(e0, axis=-1).astype(jnp.int32)
    es = jnp.take_along_axis(e0, sort_idx, axis=-1)
    rank = jnp.arange(B)[None, :] - jnp.take_along_axis(so, es, axis=-1)
    dst = jnp.take_along_axis(po, es, axis=-1) + rank
    ids_padded = jnp.zeros((3, P), jnp.int32)
    ids_padded = jax.vmap(lambda i, s, d: i.at[d].set(s))(ids_padded, sort_idx, dst)
    pos = jax.vmap(lambda p, s, d: p.at[s].set(d))(
        jnp.zeros((3, B), jnp.int32), sort_idx, dst)

    xs = jnp.take(xf, ids_padded.reshape(-1), axis=0).reshape(3, P, D)
    ys = _grouped_ffn(po.astype(jnp.int32), ntiles.astype(jnp.int32),
                      xs, W1, b1.reshape(3, E, 1, HID), W2, b2.reshape(3, E, 1, D))
    y = jnp.take_along_axis(ys, pos[..., None], axis=1)

    funcs = y.reshape(3, B, 1, 1, H, Wd)
    transform_f, add_f, quad_f = funcs[0], funcs[1], funcs[2]
    output = input * input * quad_f + input * transform_f + add_f
    output = jax.nn.sigmoid(output)
    return output, aux


# speedup vs baseline: 1.5332x; 1.2815x over previous
import functools

import jax
import jax.numpy as jnp
import numpy as np
from jax.experimental import pallas as pl
from jax.experimental.pallas import tpu as pltpu
from jax.scipy.stats import norm

B, T, H, Wd = 128, 16, 32, 32
LPAD = 128
LT = H * Wd + 2 * LPAD
_DELTAS = [Wd * (dh - 1) + (dw - 1) for dh in range(3) for dw in range(3)]
_DWS = [dw for dh in range(3) for dw in range(3)]
_CB = 8
D = H * Wd
HID = 2 * D
E = 16
NOISE_EPS = 1e-2
LOSS_COEF = 1e-2
P = 256
TILE = 8


_WMASK = np.zeros((3, H * Wd), np.float32)
for _dw in range(3):
    for _r in range(H * Wd):
        if 0 <= (_r % Wd) + _dw - 1 < Wd:
            _WMASK[_dw, _r] = 1.0
del _dw, _r


def _build_w1mt(conv1_w):
    v = jnp.transpose(conv1_w[:, 0], (0, 2, 3, 1)).reshape(10, 9, 3)
    f = jnp.concatenate([v, jnp.zeros((10, 9, 16), jnp.float32)], axis=-1)
    f = jnp.tile(f, (1, 1, 16))[:, :, :16 * 18].reshape(10, 9, 16, 18)
    return jnp.transpose(f, (0, 2, 1, 3)).reshape(160, 162)


def _conv_body(a_ref, w1_ref, b1_ref, w2_ref, wm_ref, b2_ref, out_ref):
    w1mt = w1_ref[:]
    b1col = b1_ref[:]
    w2gt = w2_ref[:]
    b2 = b2_ref[0, 0]
    masks = [wm_ref[dw:dw + 1, :] for dw in range(3)]
    for i in range(_CB):
        a = a_ref[i]
        parts = []
        for s in range(9):
            sl = a[:, LPAD + _DELTAS[s]:LPAD + _DELTAS[s] + D]
            if _DWS[s] != 1:
                sl = sl * masks[_DWS[s]]
            parts.append(sl)
        pt = jnp.concatenate(parts, axis=0)
        r = jnp.dot(w1mt, pt, preferred_element_type=jnp.float32) + b1col
        r = jnp.maximum(r, 0.0)
        g = jnp.dot(w2gt, r, preferred_element_type=jnp.float32)
        gp = jnp.pad(g, ((0, 0), (LPAD, LPAD)))
        acc = None
        for s in range(9):
            term = gp[s:s + 1, LPAD + _DELTAS[s]:LPAD + _DELTAS[s] + D]
            if _DWS[s] != 1:
                term = term * masks[_DWS[s]]
            acc = term if acc is None else acc + term
        out_ref[i, :] = jnp.maximum(acc + b2, 0.0)[0]


def _conv_front(a_t, w1mt, b1col, w2gt, wmask, b2s):
    return pl.pallas_call(
        _conv_body,
        grid=(B // _CB,),
        in_specs=[
            pl.BlockSpec((_CB, 18, LT), lambda i: (i, 0, 0)),
            pl.BlockSpec((160, 162), lambda i: (0, 0)),
            pl.BlockSpec((160, 1), lambda i: (0, 0)),
            pl.BlockSpec((9, 160), lambda i: (0, 0)),
            pl.BlockSpec((3, D), lambda i: (0, 0)),
            pl.BlockSpec((1, 1), lambda i: (0, 0), memory_space=pltpu.SMEM),
        ],
        out_specs=pl.BlockSpec((_CB, D), lambda i: (i, 0)),
        out_shape=jax.ShapeDtypeStruct((B, D), jnp.float32),
    )(a_t, w1mt, b1col, w2gt, wmask, b2s)


def _ffn_body(po_ref, nt_ref, xs_ref, w1_ref, b1_ref, w2_ref, b2_ref, out_ref):
    m = pl.program_id(0)
    e = pl.program_id(1)
    po = po_ref[m, e]
    nt = nt_ref[m, e]
    w1 = w1_ref[0, 0]
    w2 = w2_ref[0, 0]
    b1 = b1_ref[0, 0, 0]
    b2 = b2_ref[0, 0, 0]

    def tile_step(t, carry):
        base = pl.multiple_of(po + t * TILE, TILE)
        xt = xs_ref[0, pl.ds(base, TILE), :]
        h = jnp.dot(xt, w1, preferred_element_type=jnp.float32) + b1[None, :]
        h = jnp.maximum(h, 0.0)
        yt = jnp.dot(h, w2, preferred_element_type=jnp.float32) + b2[None, :]
        out_ref[0, pl.ds(base, TILE), :] = yt
        return carry

    jax.lax.fori_loop(0, nt, tile_step, 0)


def _grouped_ffn(po, nt, xs, W1, b1, W2, b2):
    grid_spec = pltpu.PrefetchScalarGridSpec(
        num_scalar_prefetch=2,
        grid=(3, E),
        in_specs=[
            pl.BlockSpec((1, P, D), lambda m, e, po, nt: (m, 0, 0)),
            pl.BlockSpec((1, 1, D, HID), lambda m, e, po, nt: (m, e, 0, 0)),
            pl.BlockSpec((1, 1, 1, HID), lambda m, e, po, nt: (m, e, 0, 0)),
            pl.BlockSpec((1, 1, HID, D), lambda m, e, po, nt: (m, e, 0, 0)),
            pl.BlockSpec((1, 1, 1, D), lambda m, e, po, nt: (m, e, 0, 0)),
        ],
        out_specs=pl.BlockSpec((1, P, D), lambda m, e, po, nt: (m, 0, 0)),
    )
    return pl.pallas_call(
        _ffn_body,
        grid_spec=grid_spec,
        out_shape=jax.ShapeDtypeStruct((3, P, D), jnp.float32),
        compiler_params=pltpu.CompilerParams(
            dimension_semantics=("arbitrary", "arbitrary"),
        ),
    )(po, nt, xs, W1, b1, W2, b2)


def _conv3d(x, w, b, padding):
    out = jax.lax.conv_general_dilated(
        x, w, window_strides=(1, 1, 1), padding=padding,
        dimension_numbers=('NCDHW', 'OIDHW', 'NCDHW'))
    return out + b[None, :, None, None, None]


def _cv_squared(x):
    eps = 1e-10
    return jnp.var(x, ddof=1) / (jnp.mean(x) ** 2 + eps)


def kernel(input, conv1_w, conv1_b, conv2_w, conv2_b, w_gate, w_noise, W1, b1, W2, b2):
    a_t = jnp.pad(input.reshape(B, T, D), ((0, 0), (1, 1), (LPAD, LPAD)))
    w1mt = _build_w1mt(conv1_w)
    b1col = jnp.repeat(conv1_b, 16).reshape(160, 1)
    w2gt = jnp.transpose(conv2_w[0], (2, 3, 0, 1)).reshape(9, 160)
    xf = _conv_front(a_t, w1mt, b1col, w2gt, jnp.asarray(_WMASK),
                     conv2_b.reshape(1, 1))

    keys = jax.random.split(jax.random.key(42), 3)
    noise = jnp.stack([jax.random.normal(k, (B, E), dtype=jnp.float32) for k in keys])

    clean = jnp.einsum('bd,mde->mbe', xf, w_gate)
    raw = jnp.einsum('bd,mde->mbe', xf, w_noise)
    stddev = jax.nn.softplus(raw) + NOISE_EPS
    noisy = clean + noise * stddev

    v0 = jnp.max(noisy, axis=-1)
    e0 = jnp.argmax(noisy, axis=-1).astype(jnp.int32)
    masked = jnp.where(jax.nn.one_hot(e0, E, dtype=bool), -jnp.inf, noisy)
    v1 = jnp.max(masked, axis=-1)

    is_in = noisy > v1[..., None]
    p_in = norm.cdf((clean - v1[..., None]) / stddev)
    p_out = norm.cdf((clean - v0[..., None]) / stddev)
    load = jnp.where(is_in, p_in, p_out).sum(axis=1)
    importance = jax.nn.one_hot(e0, E, dtype=jnp.float32).sum(axis=1)
    aux = jnp.float32(0.0)
    for m in range(3):
        aux = aux + (_cv_squared(importance[m]) + _cv_squared(load[m])) * LOSS_COEF

    counts = importance.astype(jnp.int32)
    ntiles = (counts + TILE - 1) // TILE
    po = (jnp.cumsum(ntiles, axis=-1) - ntiles) * TILE
    eq = e0[:, :, None] == e0[:, None, :]
    tri = jnp.tril(jnp.ones((B, B), bool), -1)
    rank = jnp.sum(eq & tri[None], axis=-1).astype(jnp.int32)
    slot = jnp.take_along_axis(po, e0, axis=-1) + rank
    onehot = slot[:, :, None] == jnp.arange(P)[None, None, :]
    ids_padded = jnp.sum(
        jnp.where(onehot, jnp.arange(B, dtype=jnp.int32)[None, :, None], 0),
        axis=1)

    xs = jnp.take(xf, ids_padded.reshape(-1), axis=0).reshape(3, P, D)
    ys = _grouped_ffn(po.astype(jnp.int32), ntiles.astype(jnp.int32),
                      xs, W1, b1.reshape(3, E, 1, HID), W2, b2.reshape(3, E, 1, D))
    y = jnp.take_along_axis(ys, slot[..., None], axis=1)

    funcs = y.reshape(3, B, 1, 1, H, Wd)
    transform_f, add_f, quad_f = funcs[0], funcs[1], funcs[2]
    output = input * input * quad_f + input * transform_f + add_f
    output = jax.nn.sigmoid(output)
    return output, aux


# speedup vs baseline: 1.5920x; 1.0384x over previous
import functools

import jax
import jax.numpy as jnp
import numpy as np
from jax.experimental import pallas as pl
from jax.experimental.pallas import tpu as pltpu
from jax.scipy.stats import norm

B, T, H, Wd = 128, 16, 32, 32
LPAD = 128
LT = H * Wd + 2 * LPAD
_DELTAS = [Wd * (dh - 1) + (dw - 1) for dh in range(3) for dw in range(3)]
_DWS = [dw for dh in range(3) for dw in range(3)]
_CB = 8
D = H * Wd
HID = 2 * D
E = 16
NOISE_EPS = 1e-2
LOSS_COEF = 1e-2
P = 256
TILE = 8


_WMASK = np.zeros((3, H * Wd), np.float32)
for _dw in range(3):
    for _r in range(H * Wd):
        if 0 <= (_r % Wd) + _dw - 1 < Wd:
            _WMASK[_dw, _r] = 1.0
del _dw, _r


def _build_w1mt(conv1_w):
    v = jnp.transpose(conv1_w[:, 0], (0, 2, 3, 1)).reshape(10, 9, 3)
    f = jnp.concatenate([v, jnp.zeros((10, 9, 16), jnp.float32)], axis=-1)
    f = jnp.tile(f, (1, 1, 16))[:, :, :16 * 18].reshape(10, 9, 16, 18)
    return jnp.transpose(f, (0, 2, 1, 3)).reshape(160, 162)


def _conv_body(a_ref, w1_ref, b1_ref, w2_ref, wm_ref, b2_ref, out_ref):
    w1mt = w1_ref[:]
    b1col = b1_ref[:]
    w2gt = w2_ref[:]
    b2 = b2_ref[0, 0]
    masks = [wm_ref[dw:dw + 1, :] for dw in range(3)]
    for i in range(_CB):
        a = a_ref[i]
        parts = []
        for s in range(9):
            sl = a[:, LPAD + _DELTAS[s]:LPAD + _DELTAS[s] + D]
            if _DWS[s] != 1:
                sl = sl * masks[_DWS[s]]
            parts.append(sl)
        pt = jnp.concatenate(parts, axis=0)
        r = jnp.dot(w1mt, pt, preferred_element_type=jnp.float32) + b1col
        r = jnp.maximum(r, 0.0)
        g = jnp.dot(w2gt, r, preferred_element_type=jnp.float32)
        gp = jnp.pad(g, ((0, 0), (LPAD, LPAD)))
        acc = None
        for s in range(9):
            term = gp[s:s + 1, LPAD + _DELTAS[s]:LPAD + _DELTAS[s] + D]
            if _DWS[s] != 1:
                term = term * masks[_DWS[s]]
            acc = term if acc is None else acc + term
        out_ref[i, :] = jnp.maximum(acc + b2, 0.0)[0]


def _conv_front(a_t, w1mt, b1col, w2gt, wmask, b2s):
    return pl.pallas_call(
        _conv_body,
        grid=(B // _CB,),
        in_specs=[
            pl.BlockSpec((_CB, 18, LT), lambda i: (i, 0, 0)),
            pl.BlockSpec((160, 162), lambda i: (0, 0)),
            pl.BlockSpec((160, 1), lambda i: (0, 0)),
            pl.BlockSpec((9, 160), lambda i: (0, 0)),
            pl.BlockSpec((3, D), lambda i: (0, 0)),
            pl.BlockSpec((1, 1), lambda i: (0, 0), memory_space=pltpu.SMEM),
        ],
        out_specs=pl.BlockSpec((_CB, D), lambda i: (i, 0)),
        out_shape=jax.ShapeDtypeStruct((B, D), jnp.float32),
    )(a_t, w1mt, b1col, w2gt, wmask, b2s)


def _route_body(xf_ref, wgn_ref, nz_ref, po_ref, nt_ref, slot_ref, ids_ref, aux_ref):
    xf = xf_ref[:]
    cn = jnp.dot(xf, wgn_ref[:], preferred_element_type=jnp.float32)
    lane16 = jax.lax.broadcasted_iota(jnp.int32, (B, E), 1).astype(jnp.float32)
    lane256 = jax.lax.broadcasted_iota(jnp.int32, (B, P), 1).astype(jnp.float32)
    rowb = jax.lax.broadcasted_iota(jnp.int32, (B, B), 0).astype(jnp.float32)
    colb = jax.lax.broadcasted_iota(jnp.int32, (B, B), 1).astype(jnp.float32)
    tril = colb < rowb
    excl = (jax.lax.broadcasted_iota(jnp.int32, (E, E), 0).astype(jnp.float32) <
            jax.lax.broadcasted_iota(jnp.int32, (E, E), 1).astype(jnp.float32)).astype(jnp.float32)
    aux = jnp.zeros((), jnp.float32)
    inv_sqrt2 = 0.7071067811865476
    for m in range(3):
        cl = cn[:, E * m:E * m + E]
        rw = cn[:, 48 + E * m:48 + E * m + E]
        std = jnp.logaddexp(rw, 0.0) + NOISE_EPS
        ns = cl + nz_ref[:, E * m:E * m + E] * std
        v0 = jnp.max(ns, axis=1, keepdims=True)
        first = jnp.min(jnp.where(ns >= v0, lane16, jnp.float32(E)), axis=1,
                        keepdims=True)
        oh = lane16 == first
        v1 = jnp.max(jnp.where(oh, -jnp.inf, ns), axis=1, keepdims=True)
        p_in = 0.5 * (1.0 + jax.lax.erf((cl - v1) / std * inv_sqrt2))
        p_out = 0.5 * (1.0 + jax.lax.erf((cl - v0) / std * inv_sqrt2))
        load = jnp.sum(jnp.where(ns > v1, p_in, p_out), axis=0)
        imp = jnp.sum(oh.astype(jnp.float32), axis=0)
        for v in (imp, load):
            mean = jnp.sum(v) / E
            var = jnp.sum((v - mean) ** 2) / (E - 1)
            aux = aux + var / (mean * mean + 1e-10) * LOSS_COEF
        nt = jnp.floor((imp + 7.0) / 8.0)
        po = jnp.dot(nt.reshape(1, E), excl,
                     preferred_element_type=jnp.float32)[0] * TILE
        po_ref[m, :] = po.astype(jnp.int32)
        nt_ref[m, :] = nt.astype(jnp.int32)
        e0col = jnp.sum(jnp.where(oh, lane16, 0.0), axis=1, keepdims=True)
        eqm = jnp.abs(e0col - jnp.transpose(e0col)) < 0.5
        rank = jnp.sum((eqm & tril).astype(jnp.float32), axis=1, keepdims=True)
        po_tok = jnp.dot(oh.astype(jnp.float32), po.reshape(E, 1),
                         preferred_element_type=jnp.float32)
        slot = po_tok + rank
        slot_ref[m, :] = slot[:, 0].astype(jnp.int32)
        oh256 = jnp.abs(slot - lane256) < 0.5
        rowidx = jax.lax.broadcasted_iota(jnp.int32, (B, P), 0).astype(jnp.float32)
        ids_ref[m, :] = jnp.sum(jnp.where(oh256, rowidx, 0.0),
                                axis=0).astype(jnp.int32)
    aux_ref[0, 0] = aux


def _route(xf, wgn, nz):
    return pl.pallas_call(
        _route_body,
        grid=(1,),
        in_specs=[
            pl.BlockSpec((B, D), lambda i: (0, 0)),
            pl.BlockSpec((D, 96), lambda i: (0, 0)),
            pl.BlockSpec((B, 48), lambda i: (0, 0)),
        ],
        out_specs=[
            pl.BlockSpec((3, E), lambda i: (0, 0)),
            pl.BlockSpec((3, E), lambda i: (0, 0)),
            pl.BlockSpec((3, B), lambda i: (0, 0)),
            pl.BlockSpec((3, P), lambda i: (0, 0)),
            pl.BlockSpec((1, 1), lambda i: (0, 0), memory_space=pltpu.SMEM),
        ],
        out_shape=[
            jax.ShapeDtypeStruct((3, E), jnp.int32),
            jax.ShapeDtypeStruct((3, E), jnp.int32),
            jax.ShapeDtypeStruct((3, B), jnp.int32),
            jax.ShapeDtypeStruct((3, P), jnp.int32),
            jax.ShapeDtypeStruct((1, 1), jnp.float32),
        ],
    )(xf, wgn, nz)


def _ffn_body(po_ref, nt_ref, xs_ref, w1_ref, b1_ref, w2_ref, b2_ref, out_ref):
    m = pl.program_id(0)
    e = pl.program_id(1)
    po = po_ref[m, e]
    nt = nt_ref[m, e]
    w1 = w1_ref[0, 0]
    w2 = w2_ref[0, 0]
    b1 = b1_ref[0, 0, 0]
    b2 = b2_ref[0, 0, 0]

    def tile_step(t, carry):
        base = pl.multiple_of(po + t * TILE, TILE)
        xt = xs_ref[0, pl.ds(base, TILE), :]
        h = jnp.dot(xt, w1, preferred_element_type=jnp.float32) + b1[None, :]
        h = jnp.maximum(h, 0.0)
        yt = jnp.dot(h, w2, preferred_element_type=jnp.float32) + b2[None, :]
        out_ref[0, pl.ds(base, TILE), :] = yt
        return carry

    jax.lax.fori_loop(0, nt, tile_step, 0)


def _grouped_ffn(po, nt, xs, W1, b1, W2, b2):
    grid_spec = pltpu.PrefetchScalarGridSpec(
        num_scalar_prefetch=2,
        grid=(3, E),
        in_specs=[
            pl.BlockSpec((1, P, D), lambda m, e, po, nt: (m, 0, 0)),
            pl.BlockSpec((1, 1, D, HID), lambda m, e, po, nt: (m, e, 0, 0)),
            pl.BlockSpec((1, 1, 1, HID), lambda m, e, po, nt: (m, e, 0, 0)),
            pl.BlockSpec((1, 1, HID, D), lambda m, e, po, nt: (m, e, 0, 0)),
            pl.BlockSpec((1, 1, 1, D), lambda m, e, po, nt: (m, e, 0, 0)),
        ],
        out_specs=pl.BlockSpec((1, P, D), lambda m, e, po, nt: (m, 0, 0)),
    )
    return pl.pallas_call(
        _ffn_body,
        grid_spec=grid_spec,
        out_shape=jax.ShapeDtypeStruct((3, P, D), jnp.float32),
        compiler_params=pltpu.CompilerParams(
            dimension_semantics=("arbitrary", "arbitrary"),
        ),
    )(po, nt, xs, W1, b1, W2, b2)


def _conv3d(x, w, b, padding):
    out = jax.lax.conv_general_dilated(
        x, w, window_strides=(1, 1, 1), padding=padding,
        dimension_numbers=('NCDHW', 'OIDHW', 'NCDHW'))
    return out + b[None, :, None, None, None]


def _cv_squared(x):
    eps = 1e-10
    return jnp.var(x, ddof=1) / (jnp.mean(x) ** 2 + eps)


def kernel(input, conv1_w, conv1_b, conv2_w, conv2_b, w_gate, w_noise, W1, b1, W2, b2):
    a_t = jnp.pad(input.reshape(B, T, D), ((0, 0), (1, 1), (LPAD, LPAD)))
    w1mt = _build_w1mt(conv1_w)
    b1col = jnp.repeat(conv1_b, 16).reshape(160, 1)
    w2gt = jnp.transpose(conv2_w[0], (2, 3, 0, 1)).reshape(9, 160)
    xf = _conv_front(a_t, w1mt, b1col, w2gt, jnp.asarray(_WMASK),
                     conv2_b.reshape(1, 1))

    keys = jax.random.split(jax.random.key(42), 3)
    noise = jnp.stack([jax.random.normal(k, (B, E), dtype=jnp.float32) for k in keys])

    wgn = jnp.concatenate([
        jnp.transpose(w_gate, (1, 0, 2)).reshape(D, 48),
        jnp.transpose(w_noise, (1, 0, 2)).reshape(D, 48)], axis=1)
    nz = jnp.transpose(noise, (1, 0, 2)).reshape(B, 48)
    po, ntiles, slot, ids_padded, aux_arr = _route(xf, wgn, nz)
    aux = aux_arr[0, 0]

    xs = jnp.take(xf, ids_padded.reshape(-1), axis=0).reshape(3, P, D)
    ys = _grouped_ffn(po, ntiles,
                      xs, W1, b1.reshape(3, E, 1, HID), W2, b2.reshape(3, E, 1, D))
    y = jnp.take_along_axis(ys, slot[..., None], axis=1)

    funcs = y.reshape(3, B, 1, 1, H, Wd)
    transform_f, add_f, quad_f = funcs[0], funcs[1], funcs[2]
    output = input * input * quad_f + input * transform_f + add_f
    output = jax.nn.sigmoid(output)
    return output, aux


# speedup vs baseline: 1.7093x; 1.0737x over previous
import functools

import jax
import jax.numpy as jnp
import numpy as np
from jax.experimental import pallas as pl
from jax.experimental.pallas import tpu as pltpu
from jax.scipy.stats import norm

B, T, H, Wd = 128, 16, 32, 32
LPAD = 128
LT = H * Wd + 2 * LPAD
_DELTAS = [Wd * (dh - 1) + (dw - 1) for dh in range(3) for dw in range(3)]
_DWS = [dw for dh in range(3) for dw in range(3)]
_CB = 8
D = H * Wd
HID = 2 * D
E = 16
NOISE_EPS = 1e-2
LOSS_COEF = 1e-2
P = 256
TILE = 8


_WMASK = np.zeros((3, H * Wd), np.float32)
for _dw in range(3):
    for _r in range(H * Wd):
        if 0 <= (_r % Wd) + _dw - 1 < Wd:
            _WMASK[_dw, _r] = 1.0
del _dw, _r


def _build_w1mt(conv1_w):
    v = jnp.transpose(conv1_w[:, 0], (0, 2, 3, 1)).reshape(10, 9, 3)
    f = jnp.concatenate([v, jnp.zeros((10, 9, 16), jnp.float32)], axis=-1)
    f = jnp.tile(f, (1, 1, 16))[:, :, :16 * 18].reshape(10, 9, 16, 18)
    return jnp.transpose(f, (0, 2, 1, 3)).reshape(160, 162)


def _conv_body(a_ref, w1_ref, b1_ref, w2_ref, wm_ref, b2_ref, out_ref):
    b2 = b2_ref[0, 0]
    masks = [wm_ref[dw:dw + 1, :] for dw in range(3)]
    pts = []
    for i in range(_CB):
        a = a_ref[i]
        parts = []
        for s in range(9):
            sl = a[:, LPAD + _DELTAS[s]:LPAD + _DELTAS[s] + D]
            if _DWS[s] != 1:
                sl = sl * masks[_DWS[s]]
            parts.append(sl)
        pts.append(jnp.concatenate(parts, axis=0))
    pt8 = jnp.concatenate(pts, axis=1)
    r8 = jnp.dot(w1_ref[:], pt8, preferred_element_type=jnp.float32) + b1_ref[:]
    r8 = jnp.maximum(r8, 0.0)
    g8 = jnp.dot(w2_ref[:], r8, preferred_element_type=jnp.float32)
    for i in range(_CB):
        g = g8[:, i * D:(i + 1) * D]
        gp = jnp.pad(g, ((0, 0), (LPAD, LPAD)))
        acc = None
        for s in range(9):
            term = gp[s:s + 1, LPAD + _DELTAS[s]:LPAD + _DELTAS[s] + D]
            if _DWS[s] != 1:
                term = term * masks[_DWS[s]]
            acc = term if acc is None else acc + term
        out_ref[i, :] = jnp.maximum(acc + b2, 0.0)[0]


def _conv_front(a_t, w1mt, b1col, w2gt, wmask, b2s):
    return pl.pallas_call(
        _conv_body,
        grid=(B // _CB,),
        in_specs=[
            pl.BlockSpec((_CB, 18, LT), lambda i: (i, 0, 0)),
            pl.BlockSpec((160, 162), lambda i: (0, 0)),
            pl.BlockSpec((160, 1), lambda i: (0, 0)),
            pl.BlockSpec((9, 160), lambda i: (0, 0)),
            pl.BlockSpec((3, D), lambda i: (0, 0)),
            pl.BlockSpec((1, 1), lambda i: (0, 0), memory_space=pltpu.SMEM),
        ],
        out_specs=pl.BlockSpec((_CB, D), lambda i: (i, 0)),
        out_shape=jax.ShapeDtypeStruct((B, D), jnp.float32),
    )(a_t, w1mt, b1col, w2gt, wmask, b2s)


def _route_body(xf_ref, wgn_ref, nz_ref, po_ref, nt_ref, slot_ref, ids_ref, aux_ref):
    xf = xf_ref[:]
    cn = jnp.dot(xf, wgn_ref[:], preferred_element_type=jnp.float32)
    lane16 = jax.lax.broadcasted_iota(jnp.int32, (B, E), 1).astype(jnp.float32)
    lane256 = jax.lax.broadcasted_iota(jnp.int32, (B, P), 1).astype(jnp.float32)
    rowb = jax.lax.broadcasted_iota(jnp.int32, (B, B), 0).astype(jnp.float32)
    colb = jax.lax.broadcasted_iota(jnp.int32, (B, B), 1).astype(jnp.float32)
    tril = colb < rowb
    excl = (jax.lax.broadcasted_iota(jnp.int32, (E, E), 0).astype(jnp.float32) <
            jax.lax.broadcasted_iota(jnp.int32, (E, E), 1).astype(jnp.float32)).astype(jnp.float32)
    aux = jnp.zeros((), jnp.float32)
    inv_sqrt2 = 0.7071067811865476
    for m in range(3):
        cl = cn[:, E * m:E * m + E]
        rw = cn[:, 48 + E * m:48 + E * m + E]
        std = jnp.logaddexp(rw, 0.0) + NOISE_EPS
        ns = cl + nz_ref[:, E * m:E * m + E] * std
        v0 = jnp.max(ns, axis=1, keepdims=True)
        first = jnp.min(jnp.where(ns >= v0, lane16, jnp.float32(E)), axis=1,
                        keepdims=True)
        oh = lane16 == first
        v1 = jnp.max(jnp.where(oh, -jnp.inf, ns), axis=1, keepdims=True)
        p_in = 0.5 * (1.0 + jax.lax.erf((cl - v1) / std * inv_sqrt2))
        p_out = 0.5 * (1.0 + jax.lax.erf((cl - v0) / std * inv_sqrt2))
        load = jnp.sum(jnp.where(ns > v1, p_in, p_out), axis=0)
        imp = jnp.sum(oh.astype(jnp.float32), axis=0)
        for v in (imp, load):
            mean = jnp.sum(v) / E
            var = jnp.sum((v - mean) ** 2) / (E - 1)
            aux = aux + var / (mean * mean + 1e-10) * LOSS_COEF
        nt = jnp.floor((imp + 7.0) / 8.0)
        po = jnp.dot(nt.reshape(1, E), excl,
                     preferred_element_type=jnp.float32)[0] * TILE
        po_ref[m, :] = po.astype(jnp.int32)
        nt_ref[m, :] = nt.astype(jnp.int32)
        e0col = jnp.sum(jnp.where(oh, lane16, 0.0), axis=1, keepdims=True)
        eqm = jnp.abs(e0col - jnp.transpose(e0col)) < 0.5
        rank = jnp.sum((eqm & tril).astype(jnp.float32), axis=1, keepdims=True)
        po_tok = jnp.dot(oh.astype(jnp.float32), po.reshape(E, 1),
                         preferred_element_type=jnp.float32)
        slot = po_tok + rank
        slot_ref[m, :] = slot[:, 0].astype(jnp.int32)
        oh256 = jnp.abs(slot - lane256) < 0.5
        rowidx = jax.lax.broadcasted_iota(jnp.int32, (B, P), 0).astype(jnp.float32)
        ids_ref[m, :] = jnp.sum(jnp.where(oh256, rowidx, 0.0),
                                axis=0).astype(jnp.int32)
    aux_ref[0, 0] = aux


def _route(xf, wgn, nz):
    return pl.pallas_call(
        _route_body,
        grid=(1,),
        in_specs=[
            pl.BlockSpec((B, D), lambda i: (0, 0)),
            pl.BlockSpec((D, 96), lambda i: (0, 0)),
            pl.BlockSpec((B, 48), lambda i: (0, 0)),
        ],
        out_specs=[
            pl.BlockSpec((3, E), lambda i: (0, 0)),
            pl.BlockSpec((3, E), lambda i: (0, 0)),
            pl.BlockSpec((3, B), lambda i: (0, 0)),
            pl.BlockSpec((3, P), lambda i: (0, 0)),
            pl.BlockSpec((1, 1), lambda i: (0, 0), memory_space=pltpu.SMEM),
        ],
        out_shape=[
            jax.ShapeDtypeStruct((3, E), jnp.int32),
            jax.ShapeDtypeStruct((3, E), jnp.int32),
            jax.ShapeDtypeStruct((3, B), jnp.int32),
            jax.ShapeDtypeStruct((3, P), jnp.int32),
            jax.ShapeDtypeStruct((1, 1), jnp.float32),
        ],
    )(xf, wgn, nz)


def _ffn_body(po_ref, nt_ref, xs_ref, w1_ref, b1_ref, w2_ref, b2_ref, out_ref):
    m = pl.program_id(0)
    e = pl.program_id(1)
    po = po_ref[m, e]
    nt = nt_ref[m, e]
    w1 = w1_ref[0, 0]
    w2 = w2_ref[0, 0]
    b1 = b1_ref[0, 0, 0]
    b2 = b2_ref[0, 0, 0]

    def tile_step(t, carry):
        base = pl.multiple_of(po + t * TILE, TILE)
        xt = xs_ref[0, pl.ds(base, TILE), :]
        h = jnp.dot(xt, w1, preferred_element_type=jnp.float32) + b1[None, :]
        h = jnp.maximum(h, 0.0)
        yt = jnp.dot(h, w2, preferred_element_type=jnp.float32) + b2[None, :]
        out_ref[0, pl.ds(base, TILE), :] = yt
        return carry

    jax.lax.fori_loop(0, nt, tile_step, 0)


def _grouped_ffn(po, nt, xs, W1, b1, W2, b2):
    grid_spec = pltpu.PrefetchScalarGridSpec(
        num_scalar_prefetch=2,
        grid=(3, E),
        in_specs=[
            pl.BlockSpec((1, P, D), lambda m, e, po, nt: (m, 0, 0)),
            pl.BlockSpec((1, 1, D, HID), lambda m, e, po, nt: (m, e, 0, 0)),
            pl.BlockSpec((1, 1, 1, HID), lambda m, e, po, nt: (m, e, 0, 0)),
            pl.BlockSpec((1, 1, HID, D), lambda m, e, po, nt: (m, e, 0, 0)),
            pl.BlockSpec((1, 1, 1, D), lambda m, e, po, nt: (m, e, 0, 0)),
        ],
        out_specs=pl.BlockSpec((1, P, D), lambda m, e, po, nt: (m, 0, 0)),
    )
    return pl.pallas_call(
        _ffn_body,
        grid_spec=grid_spec,
        out_shape=jax.ShapeDtypeStruct((3, P, D), jnp.float32),
        compiler_params=pltpu.CompilerParams(
            dimension_semantics=("arbitrary", "arbitrary"),
        ),
    )(po, nt, xs, W1, b1, W2, b2)


def _conv3d(x, w, b, padding):
    out = jax.lax.conv_general_dilated(
        x, w, window_strides=(1, 1, 1), padding=padding,
        dimension_numbers=('NCDHW', 'OIDHW', 'NCDHW'))
    return out + b[None, :, None, None, None]


def _cv_squared(x):
    eps = 1e-10
    return jnp.var(x, ddof=1) / (jnp.mean(x) ** 2 + eps)


def kernel(input, conv1_w, conv1_b, conv2_w, conv2_b, w_gate, w_noise, W1, b1, W2, b2):
    a_t = jnp.pad(input.reshape(B, T, D), ((0, 0), (1, 1), (LPAD, LPAD)))
    w1mt = _build_w1mt(conv1_w)
    b1col = jnp.repeat(conv1_b, 16).reshape(160, 1)
    w2gt = jnp.transpose(conv2_w[0], (2, 3, 0, 1)).reshape(9, 160)
    xf = _conv_front(a_t, w1mt, b1col, w2gt, jnp.asarray(_WMASK),
                     conv2_b.reshape(1, 1))

    keys = jax.random.split(jax.random.key(42), 3)
    noise = jnp.stack([jax.random.normal(k, (B, E), dtype=jnp.float32) for k in keys])

    wgn = jnp.concatenate([
        jnp.transpose(w_gate, (1, 0, 2)).reshape(D, 48),
        jnp.transpose(w_noise, (1, 0, 2)).reshape(D, 48)], axis=1)
    nz = jnp.transpose(noise, (1, 0, 2)).reshape(B, 48)
    po, ntiles, slot, ids_padded, aux_arr = _route(xf, wgn, nz)
    aux = aux_arr[0, 0]

    xs = jnp.take(xf, ids_padded.reshape(-1), axis=0).reshape(3, P, D)
    ys = _grouped_ffn(po, ntiles,
                      xs, W1, b1.reshape(3, E, 1, HID), W2, b2.reshape(3, E, 1, D))
    y = jnp.take_along_axis(ys, slot[..., None], axis=1)

    funcs = y.reshape(3, B, 1, 1, H, Wd)
    transform_f, add_f, quad_f = funcs[0], funcs[1], funcs[2]
    output = input * input * quad_f + input * transform_f + add_f
    output = jax.nn.sigmoid(output)
    return output, aux


# speedup vs baseline: 1.7359x; 1.0155x over previous
import functools

import jax
import jax.numpy as jnp
import numpy as np
from jax.experimental import pallas as pl
from jax.experimental.pallas import tpu as pltpu
from jax.scipy.stats import norm

B, T, H, Wd = 128, 16, 32, 32
LPAD = 128
LT = H * Wd + 2 * LPAD
_DELTAS = [Wd * (dh - 1) + (dw - 1) for dh in range(3) for dw in range(3)]
_DWS = [dw for dh in range(3) for dw in range(3)]
_CB = 16
D = H * Wd
HID = 2 * D
E = 16
NOISE_EPS = 1e-2
LOSS_COEF = 1e-2
P = 256
TILE = 8


_WMASK = np.zeros((3, H * Wd), np.float32)
for _dw in range(3):
    for _r in range(H * Wd):
        if 0 <= (_r % Wd) + _dw - 1 < Wd:
            _WMASK[_dw, _r] = 1.0
del _dw, _r


def _build_w1mt(conv1_w):
    v = jnp.transpose(conv1_w[:, 0], (0, 2, 3, 1)).reshape(10, 9, 3)
    f = jnp.concatenate([v, jnp.zeros((10, 9, 16), jnp.float32)], axis=-1)
    f = jnp.tile(f, (1, 1, 16))[:, :, :16 * 18].reshape(10, 9, 16, 18)
    return jnp.transpose(f, (0, 2, 1, 3)).reshape(160, 162)


def _conv_body(a_ref, w1_ref, b1_ref, w2_ref, wm_ref, b2_ref, out_ref):
    b2 = b2_ref[0, 0]
    masks = [wm_ref[dw:dw + 1, :] for dw in range(3)]
    pts = []
    for i in range(_CB):
        a = a_ref[i]
        parts = []
        for s in range(9):
            sl = a[:, LPAD + _DELTAS[s]:LPAD + _DELTAS[s] + D]
            if _DWS[s] != 1:
                sl = sl * masks[_DWS[s]]
            parts.append(sl)
        pts.append(jnp.concatenate(parts, axis=0))
    pt8 = jnp.concatenate(pts, axis=1)
    r8 = jnp.dot(w1_ref[:], pt8, preferred_element_type=jnp.float32) + b1_ref[:]
    r8 = jnp.maximum(r8, 0.0)
    g8 = jnp.dot(w2_ref[:], r8, preferred_element_type=jnp.float32)
    for i in range(_CB):
        g = g8[:, i * D:(i + 1) * D]
        gp = jnp.pad(g, ((0, 0), (LPAD, LPAD)))
        acc = None
        for s in range(9):
            term = gp[s:s + 1, LPAD + _DELTAS[s]:LPAD + _DELTAS[s] + D]
            if _DWS[s] != 1:
                term = term * masks[_DWS[s]]
            acc = term if acc is None else acc + term
        out_ref[i, :] = jnp.maximum(acc + b2, 0.0)[0]


def _conv_front(a_t, w1mt, b1col, w2gt, wmask, b2s):
    return pl.pallas_call(
        _conv_body,
        grid=(B // _CB,),
        in_specs=[
            pl.BlockSpec((_CB, 18, LT), lambda i: (i, 0, 0)),
            pl.BlockSpec((160, 162), lambda i: (0, 0)),
            pl.BlockSpec((160, 1), lambda i: (0, 0)),
            pl.BlockSpec((9, 160), lambda i: (0, 0)),
            pl.BlockSpec((3, D), lambda i: (0, 0)),
            pl.BlockSpec((1, 1), lambda i: (0, 0), memory_space=pltpu.SMEM),
        ],
        out_specs=pl.BlockSpec((_CB, D), lambda i: (i, 0)),
        out_shape=jax.ShapeDtypeStruct((B, D), jnp.float32),
    )(a_t, w1mt, b1col, w2gt, wmask, b2s)


def _route_body(xf_ref, wgn_ref, nz_ref, po_ref, nt_ref, slot_ref, ids_ref, aux_ref):
    xf = xf_ref[:]
    cn = jnp.dot(xf, wgn_ref[:], preferred_element_type=jnp.float32)
    lane16 = jax.lax.broadcasted_iota(jnp.int32, (B, E), 1).astype(jnp.float32)
    lane256 = jax.lax.broadcasted_iota(jnp.int32, (B, P), 1).astype(jnp.float32)
    rowb = jax.lax.broadcasted_iota(jnp.int32, (B, B), 0).astype(jnp.float32)
    colb = jax.lax.broadcasted_iota(jnp.int32, (B, B), 1).astype(jnp.float32)
    tril = colb < rowb
    excl = (jax.lax.broadcasted_iota(jnp.int32, (E, E), 0).astype(jnp.float32) <
            jax.lax.broadcasted_iota(jnp.int32, (E, E), 1).astype(jnp.float32)).astype(jnp.float32)
    aux = jnp.zeros((), jnp.float32)
    inv_sqrt2 = 0.7071067811865476
    for m in range(3):
        cl = cn[:, E * m:E * m + E]
        rw = cn[:, 48 + E * m:48 + E * m + E]
        std = jnp.logaddexp(rw, 0.0) + NOISE_EPS
        ns = cl + nz_ref[:, E * m:E * m + E] * std
        v0 = jnp.max(ns, axis=1, keepdims=True)
        first = jnp.min(jnp.where(ns >= v0, lane16, jnp.float32(E)), axis=1,
                        keepdims=True)
        oh = lane16 == first
        v1 = jnp.max(jnp.where(oh, -jnp.inf, ns), axis=1, keepdims=True)
        p_in = 0.5 * (1.0 + jax.lax.erf((cl - v1) / std * inv_sqrt2))
        p_out = 0.5 * (1.0 + jax.lax.erf((cl - v0) / std * inv_sqrt2))
        load = jnp.sum(jnp.where(ns > v1, p_in, p_out), axis=0)
        imp = jnp.sum(oh.astype(jnp.float32), axis=0)
        for v in (imp, load):
            mean = jnp.sum(v) / E
            var = jnp.sum((v - mean) ** 2) / (E - 1)
            aux = aux + var / (mean * mean + 1e-10) * LOSS_COEF
        nt = jnp.floor((imp + 7.0) / 8.0)
        po = jnp.dot(nt.reshape(1, E), excl,
                     preferred_element_type=jnp.float32)[0] * TILE
        po_ref[m, :] = po.astype(jnp.int32)
        nt_ref[m, :] = nt.astype(jnp.int32)
        e0col = jnp.sum(jnp.where(oh, lane16, 0.0), axis=1, keepdims=True)
        eqm = jnp.abs(e0col - jnp.transpose(e0col)) < 0.5
        rank = jnp.sum((eqm & tril).astype(jnp.float32), axis=1, keepdims=True)
        po_tok = jnp.dot(oh.astype(jnp.float32), po.reshape(E, 1),
                         preferred_element_type=jnp.float32)
        slot = po_tok + rank
        slot_ref[m, :] = slot[:, 0].astype(jnp.int32)
        oh256 = jnp.abs(slot - lane256) < 0.5
        rowidx = jax.lax.broadcasted_iota(jnp.int32, (B, P), 0).astype(jnp.float32)
        ids_ref[m, :] = jnp.sum(jnp.where(oh256, rowidx, 0.0),
                                axis=0).astype(jnp.int32)
    aux_ref[0, 0] = aux


def _route(xf, wgn, nz):
    return pl.pallas_call(
        _route_body,
        grid=(1,),
        in_specs=[
            pl.BlockSpec((B, D), lambda i: (0, 0)),
            pl.BlockSpec((D, 96), lambda i: (0, 0)),
            pl.BlockSpec((B, 48), lambda i: (0, 0)),
        ],
        out_specs=[
            pl.BlockSpec((3, E), lambda i: (0, 0)),
            pl.BlockSpec((3, E), lambda i: (0, 0)),
            pl.BlockSpec((3, B), lambda i: (0, 0)),
            pl.BlockSpec((3, P), lambda i: (0, 0)),
            pl.BlockSpec((1, 1), lambda i: (0, 0), memory_space=pltpu.SMEM),
        ],
        out_shape=[
            jax.ShapeDtypeStruct((3, E), jnp.int32),
            jax.ShapeDtypeStruct((3, E), jnp.int32),
            jax.ShapeDtypeStruct((3, B), jnp.int32),
            jax.ShapeDtypeStruct((3, P), jnp.int32),
            jax.ShapeDtypeStruct((1, 1), jnp.float32),
        ],
    )(xf, wgn, nz)


def _ffn_body(po_ref, nt_ref, xs_ref, w1_ref, b1_ref, w2_ref, b2_ref, out_ref):
    m = pl.program_id(0)
    e = pl.program_id(1)
    po = po_ref[m, e]
    nt = nt_ref[m, e]
    w1 = w1_ref[0, 0]
    w2 = w2_ref[0, 0]
    b1 = b1_ref[0, 0, 0]
    b2 = b2_ref[0, 0, 0]

    def tile_step(t, carry):
        base = pl.multiple_of(po + t * TILE, TILE)
        xt = xs_ref[0, pl.ds(base, TILE), :]
        h = jnp.dot(xt, w1, preferred_element_type=jnp.float32) + b1[None, :]
        h = jnp.maximum(h, 0.0)
        yt = jnp.dot(h, w2, preferred_element_type=jnp.float32) + b2[None, :]
        out_ref[0, pl.ds(base, TILE), :] = yt
        return carry

    jax.lax.fori_loop(0, nt, tile_step, 0)


def _grouped_ffn(po, nt, xs, W1, b1, W2, b2):
    grid_spec = pltpu.PrefetchScalarGridSpec(
        num_scalar_prefetch=2,
        grid=(3, E),
        in_specs=[
            pl.BlockSpec((1, P, D), lambda m, e, po, nt: (m, 0, 0)),
            pl.BlockSpec((1, 1, D, HID), lambda m, e, po, nt: (m, e, 0, 0)),
            pl.BlockSpec((1, 1, 1, HID), lambda m, e, po, nt: (m, e, 0, 0)),
            pl.BlockSpec((1, 1, HID, D), lambda m, e, po, nt: (m, e, 0, 0)),
            pl.BlockSpec((1, 1, 1, D), lambda m, e, po, nt: (m, e, 0, 0)),
        ],
        out_specs=pl.BlockSpec((1, P, D), lambda m, e, po, nt: (m, 0, 0)),
    )
    return pl.pallas_call(
        _ffn_body,
        grid_spec=grid_spec,
        out_shape=jax.ShapeDtypeStruct((3, P, D), jnp.float32),
        compiler_params=pltpu.CompilerParams(
            dimension_semantics=("arbitrary", "arbitrary"),
        ),
    )(po, nt, xs, W1, b1, W2, b2)


def _conv3d(x, w, b, padding):
    out = jax.lax.conv_general_dilated(
        x, w, window_strides=(1, 1, 1), padding=padding,
        dimension_numbers=('NCDHW', 'OIDHW', 'NCDHW'))
    return out + b[None, :, None, None, None]


def _cv_squared(x):
    eps = 1e-10
    return jnp.var(x, ddof=1) / (jnp.mean(x) ** 2 + eps)


def kernel(input, conv1_w, conv1_b, conv2_w, conv2_b, w_gate, w_noise, W1, b1, W2, b2):
    a_t = jnp.pad(input.reshape(B, T, D), ((0, 0), (1, 1), (LPAD, LPAD)))
    w1mt = _build_w1mt(conv1_w)
    b1col = jnp.repeat(conv1_b, 16).reshape(160, 1)
    w2gt = jnp.transpose(conv2_w[0], (2, 3, 0, 1)).reshape(9, 160)
    xf = _conv_front(a_t, w1mt, b1col, w2gt, jnp.asarray(_WMASK),
                     conv2_b.reshape(1, 1))

    keys = jax.random.split(jax.random.key(42), 3)
    noise = jnp.stack([jax.random.normal(k, (B, E), dtype=jnp.float32) for k in keys])

    wgn = jnp.concatenate([
        jnp.transpose(w_gate, (1, 0, 2)).reshape(D, 48),
        jnp.transpose(w_noise, (1, 0, 2)).reshape(D, 48)], axis=1)
    nz = jnp.transpose(noise, (1, 0, 2)).reshape(B, 48)
    po, ntiles, slot, ids_padded, aux_arr = _route(xf, wgn, nz)
    aux = aux_arr[0, 0]

    xs = jnp.take(xf, ids_padded.reshape(-1), axis=0).reshape(3, P, D)
    ys = _grouped_ffn(po, ntiles,
                      xs, W1, b1.reshape(3, E, 1, HID), W2, b2.reshape(3, E, 1, D))
    y = jnp.take_along_axis(ys, slot[..., None], axis=1)

    funcs = y.reshape(3, B, 1, 1, H, Wd)
    transform_f, add_f, quad_f = funcs[0], funcs[1], funcs[2]
    output = input * input * quad_f + input * transform_f + add_f
    output = jax.nn.sigmoid(output)
    return output, aux
